# baseline (speedup 1.0000x reference)
import jax
import jax.numpy as jnp
import numpy as np
from jax.experimental import pallas as pl
from jax.experimental.pallas import tpu as pltpu

N = 10000
F = 128
RATIO = 0.5
RADIUS = 0.2
MAX_NB = 64
M = int(N * RATIO)


def _fps(pos, n_samples):
    Np = pos.shape[0]

    def body(i, state):
        dists, idxs = state
        last = idxs[i - 1]
        d = jnp.sum((pos - pos[last]) ** 2, axis=1)
        dists = jnp.minimum(dists, d)
        idxs = idxs.at[i].set(jnp.argmax(dists).astype(jnp.int32))
        return (dists, idxs)

    dists = jnp.full((Np,), jnp.inf, dtype=jnp.float32)
    idxs = jnp.zeros((n_samples,), dtype=jnp.int32)
    dists, idxs = jax.lax.fori_loop(1, n_samples, body, (dists, idxs))
    return idxs


def _radius(pos, pos_c, r, max_nb):
    d2 = jnp.sum((pos_c[:, None, :] - pos[None, :, :]) ** 2, axis=-1)
    scores = jnp.where(d2 <= r * r, -d2, -jnp.inf)
    vals, nbr = jax.lax.top_k(scores, max_nb)
    valid = vals > -jnp.inf
    return nbr, valid


def _mm2_body(h_ref, w_ref, b_ref, o_ref):
    o_ref[...] = (
        jnp.dot(h_ref[...], w_ref[...], preferred_element_type=jnp.float32)
        + b_ref[...]
    )


def _mm2(h, W2, b2):
    E = h.shape[0]
    BLK = 1000
    E_pad = ((E + BLK - 1) // BLK) * BLK
    hp = jnp.pad(h, ((0, E_pad - E), (0, 0)))
    out = pl.pallas_call(
        _mm2_body,
        grid=(E_pad // BLK,),
        in_specs=[
            pl.BlockSpec((BLK, 128), lambda i: (i, 0)),
            pl.BlockSpec((128, 256), lambda i: (0, 0)),
            pl.BlockSpec((1, 256), lambda i: (0, 0)),
        ],
        out_specs=pl.BlockSpec((BLK, 256), lambda i: (i, 0)),
        out_shape=jax.ShapeDtypeStruct((E_pad, 256), jnp.float32),
    )(hp, W2, b2[None, :])
    return out[:E]


def kernel(x, pos, batch, W1, b1, gamma, beta, W2, b2):
    idx = _fps(pos, M)
    pos_c = pos[idx]
    nbr, valid = _radius(pos, pos_c, RADIUS, MAX_NB)
    col_f = nbr.reshape(-1)
    row_f = jnp.repeat(jnp.arange(M, dtype=jnp.int32), MAX_NB)
    mask_f = valid.reshape(-1) & (col_f != row_f)
    loops = jnp.arange(M, dtype=jnp.int32)
    col = jnp.concatenate([col_f, loops])
    row = jnp.concatenate([row_f, loops])
    mask = jnp.concatenate([mask_f, jnp.ones((M,), dtype=bool)])
    msg = pos[col] - pos_c[row] / RADIUS
    h = jnp.concatenate([x[col], msg], axis=1)
    h = h @ W1 + b1
    w = mask.astype(h.dtype)[:, None]
    cnt = jnp.sum(w)
    mu = jnp.sum(h * w, axis=0) / cnt
    var = jnp.sum(((h - mu) ** 2) * w, axis=0) / cnt
    h = (h - mu) / jnp.sqrt(var + 1e-5) * gamma + beta
    h = jax.nn.relu(h)
    h = _mm2(h, W2, b2)
    h = jnp.where(mask[:, None], h, -jnp.inf)
    out = jax.ops.segment_max(h, row, num_segments=M)
    return (out, pos_c, batch[idx])


# speedup vs baseline: 2.7531x; 2.7531x over previous
import jax
import jax.numpy as jnp
import numpy as np
from jax.experimental import pallas as pl
from jax.experimental.pallas import tpu as pltpu

N = 10000
F = 128
RATIO = 0.5
RADIUS = 0.2
MAX_NB = 64
M = int(N * RATIO)


NPAD = 10240
ROWS = NPAD // 128


def _fps_body(p0_ref, p1_ref, p2_ref, o_ref, dists_ref):
    BIG = jnp.int32(2**30)
    gidx = (jax.lax.broadcasted_iota(jnp.int32, (ROWS, 128), 0) * 128
            + jax.lax.broadcasted_iota(jnp.int32, (ROWS, 128), 1))
    dists_ref[...] = jnp.where(gidx < N, jnp.inf, -jnp.inf)
    o_ref[0] = 0
    p0 = p0_ref[...]
    p1 = p1_ref[...]
    p2 = p2_ref[...]

    def body(i, carry):
        px, py, pz = carry
        dx = p0 - px
        dy = p1 - py
        dz = p2 - pz
        d = (dx * dx + dy * dy) + dz * dz
        dists = jnp.minimum(dists_ref[...], d)
        dists_ref[...] = dists
        mx = jnp.max(dists)
        last = jnp.min(jnp.where(dists == mx, gidx, BIG))
        o_ref[i] = last
        sel = gidx == last
        zero = jnp.float32(0)
        npx = jnp.sum(jnp.where(sel, p0, zero))
        npy = jnp.sum(jnp.where(sel, p1, zero))
        npz = jnp.sum(jnp.where(sel, p2, zero))
        return (npx, npy, npz)

    jax.lax.fori_loop(1, M, body, (p0[0, 0], p1[0, 0], p2[0, 0]))


def _fps(pos):
    pp = jnp.pad(pos, ((0, NPAD - N), (0, 0))).T.reshape(3, ROWS, 128)
    return pl.pallas_call(
        _fps_body,
        in_specs=[
            pl.BlockSpec((ROWS, 128), lambda: (0, 0)),
            pl.BlockSpec((ROWS, 128), lambda: (0, 0)),
            pl.BlockSpec((ROWS, 128), lambda: (0, 0)),
        ],
        out_specs=pl.BlockSpec(memory_space=pltpu.SMEM),
        out_shape=jax.ShapeDtypeStruct((M,), jnp.int32),
        scratch_shapes=[pltpu.VMEM((ROWS, 128), jnp.float32)],
    )(pp[0], pp[1], pp[2])


def _radius(pos, pos_c, r, max_nb):
    d2 = jnp.sum((pos_c[:, None, :] - pos[None, :, :]) ** 2, axis=-1)
    scores = jnp.where(d2 <= r * r, -d2, -jnp.inf)
    vals, nbr = jax.lax.top_k(scores, max_nb)
    valid = vals > -jnp.inf
    return nbr, valid


def _mm2_body(h_ref, w_ref, b_ref, o_ref):
    o_ref[...] = (
        jnp.dot(h_ref[...], w_ref[...], preferred_element_type=jnp.float32)
        + b_ref[...]
    )


def _mm2(h, W2, b2):
    E = h.shape[0]
    BLK = 1000
    E_pad = ((E + BLK - 1) // BLK) * BLK
    hp = jnp.pad(h, ((0, E_pad - E), (0, 0)))
    out = pl.pallas_call(
        _mm2_body,
        grid=(E_pad // BLK,),
        in_specs=[
            pl.BlockSpec((BLK, 128), lambda i: (i, 0)),
            pl.BlockSpec((128, 256), lambda i: (0, 0)),
            pl.BlockSpec((1, 256), lambda i: (0, 0)),
        ],
        out_specs=pl.BlockSpec((BLK, 256), lambda i: (i, 0)),
        out_shape=jax.ShapeDtypeStruct((E_pad, 256), jnp.float32),
    )(hp, W2, b2[None, :])
    return out[:E]


def kernel(x, pos, batch, W1, b1, gamma, beta, W2, b2):
    idx = _fps(pos)
    pos_c = pos[idx]
    nbr, valid = _radius(pos, pos_c, RADIUS, MAX_NB)
    col_f = nbr.reshape(-1)
    row_f = jnp.repeat(jnp.arange(M, dtype=jnp.int32), MAX_NB)
    mask_f = valid.reshape(-1) & (col_f != row_f)
    loops = jnp.arange(M, dtype=jnp.int32)
    col = jnp.concatenate([col_f, loops])
    row = jnp.concatenate([row_f, loops])
    mask = jnp.concatenate([mask_f, jnp.ones((M,), dtype=bool)])
    msg = pos[col] - pos_c[row] / RADIUS
    h = jnp.concatenate([x[col], msg], axis=1)
    h = h @ W1 + b1
    w = mask.astype(h.dtype)[:, None]
    cnt = jnp.sum(w)
    mu = jnp.sum(h * w, axis=0) / cnt
    var = jnp.sum(((h - mu) ** 2) * w, axis=0) / cnt
    h = (h - mu) / jnp.sqrt(var + 1e-5) * gamma + beta
    h = jax.nn.relu(h)
    h = _mm2(h, W2, b2)
    h = jnp.where(mask[:, None], h, -jnp.inf)
    out = jax.ops.segment_max(h, row, num_segments=M)
    return (out, pos_c, batch[idx])


# speedup vs baseline: 11.8355x; 4.2990x over previous
import functools

import jax
import jax.numpy as jnp
import numpy as np
from jax import lax
from jax.experimental import pallas as pl
from jax.experimental.pallas import tpu as pltpu
from jax.experimental.pallas import tpu_sc as plsc

N = 10000
F = 128
RATIO = 0.5
RADIUS = 0.2
MAX_NB = 64
M = int(N * RATIO)

NPAD = 10240
ROWS = NPAD // 128


def _fps_body(p0_ref, p1_ref, p2_ref, o_ref, dists_ref):
    BIG = jnp.int32(2**30)
    gidx = (jax.lax.broadcasted_iota(jnp.int32, (ROWS, 128), 0) * 128
            + jax.lax.broadcasted_iota(jnp.int32, (ROWS, 128), 1))
    dists_ref[...] = jnp.where(gidx < N, jnp.inf, -jnp.inf)
    o_ref[0] = 0
    p0 = p0_ref[...]
    p1 = p1_ref[...]
    p2 = p2_ref[...]

    def body(i, carry):
        px, py, pz = carry
        dx = p0 - px
        dy = p1 - py
        dz = p2 - pz
        d = (dx * dx + dy * dy) + dz * dz
        dists = jnp.minimum(dists_ref[...], d)
        dists_ref[...] = dists
        mx = jnp.max(dists)
        last = jnp.min(jnp.where(dists == mx, gidx, BIG))
        o_ref[i] = last
        sel = gidx == last
        zero = jnp.float32(0)
        npx = jnp.sum(jnp.where(sel, p0, zero))
        npy = jnp.sum(jnp.where(sel, p1, zero))
        npz = jnp.sum(jnp.where(sel, p2, zero))
        return (npx, npy, npz)

    jax.lax.fori_loop(1, M, body, (p0[0, 0], p1[0, 0], p2[0, 0]))


def _fps(pos):
    pp = jnp.pad(pos, ((0, NPAD - N), (0, 0))).T.reshape(3, ROWS, 128)
    return pl.pallas_call(
        _fps_body,
        in_specs=[
            pl.BlockSpec((ROWS, 128), lambda: (0, 0)),
            pl.BlockSpec((ROWS, 128), lambda: (0, 0)),
            pl.BlockSpec((ROWS, 128), lambda: (0, 0)),
        ],
        out_specs=pl.BlockSpec(memory_space=pltpu.SMEM),
        out_shape=jax.ShapeDtypeStruct((M,), jnp.int32),
        scratch_shapes=[pltpu.VMEM((ROWS, 128), jnp.float32)],
    )(pp[0], pp[1], pp[2])


def _radius(pos, pos_c, r, max_nb):
    d2 = jnp.sum((pos_c[:, None, :] - pos[None, :, :]) ** 2, axis=-1)
    scores = jnp.where(d2 <= r * r, -d2, -jnp.inf)
    vals, nbr = jax.lax.top_k(scores, max_nb)
    valid = vals > -jnp.inf
    return nbr, valid


NSC = 32
WPC = 160
M_PAD = NSC * WPC
R2BITS = 1025758986
NCHUNK = N // 16


def _sc_select_body(px_h, py_h, pz_h, cx_h, cy_h, cz_h, x1p_h,
                    nbr_h, val_h, g_h,
                    px, py, pz, cx, cy, cz,
                    cd2, cidx, nbrs, vals, gbuf0, gbuf1,
                    sem_in, sem_g0, sem_g1, sem_out):
    wid = lax.axis_index("s") * 2 + lax.axis_index("c")
    base_c = wid * WPC
    pltpu.sync_copy(px_h, px)
    pltpu.sync_copy(py_h, py)
    pltpu.sync_copy(pz_h, pz)
    pltpu.sync_copy(cx_h.at[pl.ds(base_c * 16, WPC * 16)], cx)
    pltpu.sync_copy(cy_h.at[pl.ds(base_c * 16, WPC * 16)], cy)
    pltpu.sync_copy(cz_h.at[pl.ds(base_c * 16, WPC * 16)], cz)

    lane = lax.iota(jnp.int32, 16)
    r2 = jnp.float32(RADIUS * RADIUS)

    def per_centroid(c, _):
        ci = base_c + c
        ccx = cx[pl.ds(c * 16, 16)]
        ccy = cy[pl.ds(c * 16, 16)]
        ccz = cz[pl.ds(c * 16, 16)]

        def scan_body(t, off):
            dx = px[pl.ds(t * 16, 16)] - ccx
            dy = py[pl.ds(t * 16, 16)] - ccy
            dz = pz[pl.ds(t * 16, 16)] - ccz
            d2 = (dx * dx + dy * dy) + dz * dz
            m = d2 <= r2
            mi = m.astype(jnp.int32)
            dst = plsc.cumsum(mi) - mi + off
            plsc.store_scatter(cd2, [dst], d2, mask=m)
            plsc.store_scatter(cidx, [dst], lane + t * 16, mask=m)
            return off + jnp.sum(mi)

        cnt_in = lax.fori_loop(0, NCHUNK, scan_body, jnp.int32(0))
        nv = (cnt_in + 15) // 16

        def count_le(v):
            def cb(t, acc):
                d2b = plsc.bitcast(cd2[pl.ds(t * 16, 16)], jnp.int32)
                okm = (lane + t * 16) < cnt_in
                return acc + jnp.sum(((d2b <= v) & okm).astype(jnp.int32))
            return lax.fori_loop(0, nv, cb, jnp.int32(0))

        def bs(_, lohi):
            lo, hi = lohi
            mid = (lo + hi) // 2
            le = count_le(mid)
            big = le >= MAX_NB
            return (jnp.where(big, lo, mid + 1), jnp.where(big, mid, hi))

        lo, hi = lax.fori_loop(0, 31, bs, (jnp.int32(0), jnp.int32(R2BITS)))
        v64 = jnp.where(cnt_in <= MAX_NB, jnp.int32(R2BITS + 1), lo)

        def count_lt(v):
            def cb(t, acc):
                d2b = plsc.bitcast(cd2[pl.ds(t * 16, 16)], jnp.int32)
                okm = (lane + t * 16) < cnt_in
                return acc + jnp.sum(((d2b < v) & okm).astype(jnp.int32))
            return lax.fori_loop(0, nv, cb, jnp.int32(0))

        n_lt = count_lt(v64)
        need = MAX_NB - n_lt

        row = c * MAX_NB
        splat_ci = jnp.full((16,), 0, jnp.int32) + ci
        for q in range(MAX_NB // 16):
            nbrs[pl.ds(row + q * 16, 16)] = splat_ci

        def sel_body(t, carry):
            off2, ties = carry
            d2b = plsc.bitcast(cd2[pl.ds(t * 16, 16)], jnp.int32)
            idxv = cidx[pl.ds(t * 16, 16)]
            okm = (lane + t * 16) < cnt_in
            lt = (d2b < v64) & okm
            eq = (d2b == v64) & okm
            pref = plsc.cumsum(eq.astype(jnp.int32)) + ties
            sel = lt | (eq & (pref <= need))
            seli = sel.astype(jnp.int32)
            dst = plsc.cumsum(seli) - seli + (row + off2)
            plsc.store_scatter(nbrs, [dst], idxv, mask=sel)
            return (off2 + jnp.sum(seli),
                    ties + jnp.sum(eq.astype(jnp.int32)))

        s, _ = lax.fori_loop(0, nv, sel_body, (jnp.int32(0), jnp.int32(0)))

        for q in range(MAX_NB // 16):
            vals[pl.ds(row + q * 16, 16)] = ((lane + q * 16) < s).astype(jnp.int32)
        return _

    lax.fori_loop(0, WPC, per_centroid, 0)

    pltpu.sync_copy(nbrs.at[pl.ds(0, WPC * MAX_NB)],
                    nbr_h.at[pl.ds(base_c * MAX_NB, WPC * MAX_NB)])
    pltpu.sync_copy(vals.at[pl.ds(0, WPC * MAX_NB)],
                    val_h.at[pl.ds(base_c * MAX_NB, WPC * MAX_NB)])

    def gather_one(c, _):
        row = c * MAX_NB
        for q in range(MAX_NB // 16):
            idxv = nbrs[pl.ds(row + q * 16, 16)]
            pltpu.async_copy(x1p_h.at[idxv],
                             gbuf0.at[pl.ds(q * 16, 16)], sem_g0).wait()
        pltpu.sync_copy(gbuf0, g_h.at[pl.ds((base_c + c) * MAX_NB, MAX_NB)])
        return _

    lax.fori_loop(0, WPC, gather_one, 0)


def _sc_select_gather(pos, pos_c_pad, X1p):
    posp = pos.T
    crep = jnp.repeat(pos_c_pad.T.reshape(3, M_PAD), 16, axis=1)
    mesh = plsc.VectorSubcoreMesh(core_axis_name="c", subcore_axis_name="s")
    f = functools.partial(
        pl.kernel,
        mesh=mesh,
        compiler_params=pltpu.CompilerParams(needs_layout_passes=False),
        out_type=[
            jax.ShapeDtypeStruct((M_PAD * MAX_NB,), jnp.int32),
            jax.ShapeDtypeStruct((M_PAD * MAX_NB,), jnp.int32),
            jax.ShapeDtypeStruct((M_PAD * MAX_NB, 128), jnp.float32),
        ],
        scratch_types=[
            pltpu.VMEM((N,), jnp.float32),
            pltpu.VMEM((N,), jnp.float32),
            pltpu.VMEM((N,), jnp.float32),
            pltpu.VMEM((WPC * 16,), jnp.float32),
            pltpu.VMEM((WPC * 16,), jnp.float32),
            pltpu.VMEM((WPC * 16,), jnp.float32),
            pltpu.VMEM((N + 16,), jnp.float32),
            pltpu.VMEM((N + 16,), jnp.int32),
            pltpu.VMEM((WPC * MAX_NB + 16,), jnp.int32),
            pltpu.VMEM((WPC * MAX_NB + 16,), jnp.int32),
            pltpu.VMEM((MAX_NB, 128), jnp.float32),
            pltpu.VMEM((MAX_NB, 128), jnp.float32),
            pltpu.SemaphoreType.DMA,
            pltpu.SemaphoreType.DMA,
            pltpu.SemaphoreType.DMA,
            pltpu.SemaphoreType.DMA,
        ],
    )(_sc_select_body)
    nbr, val, G = f(posp[0], posp[1], posp[2],
                    crep[0], crep[1], crep[2], X1p)
    return (nbr.reshape(M_PAD, MAX_NB), val.reshape(M_PAD, MAX_NB),
            G.reshape(M_PAD, MAX_NB, 128))


def _x1p_body(xin_ref, w_ref, b_ref, o_ref):
    o_ref[...] = (
        jnp.dot(xin_ref[...], w_ref[...], preferred_element_type=jnp.float32)
        + b_ref[...]
    )


def _x1p(x, pos, W1, b1):
    xin = jnp.concatenate([x, pos, jnp.zeros((N, 5), jnp.float32)], axis=1)
    W1p = jnp.concatenate([W1, jnp.zeros((5, 128), jnp.float32)], axis=0)
    BLK = 1000
    return pl.pallas_call(
        _x1p_body,
        grid=(N // BLK,),
        in_specs=[
            pl.BlockSpec((BLK, 136), lambda i: (i, 0)),
            pl.BlockSpec((136, 128), lambda i: (0, 0)),
            pl.BlockSpec((1, 128), lambda i: (0, 0)),
        ],
        out_specs=pl.BlockSpec((BLK, 128), lambda i: (i, 0)),
        out_shape=jax.ShapeDtypeStruct((N, 128), jnp.float32),
    )(xin, W1p, b1[None, :])


BC = 8


def _stats_body(g_ref, x1s_ref, c1_ref, nbr_ref, val_ref, sh_ref, sh2_ref, cnt_ref):
    pid = pl.program_id(0)
    gi = pid * BC + jax.lax.broadcasted_iota(jnp.int32, (BC, MAX_NB), 0)
    mask = (val_ref[...] != 0) & (nbr_ref[...] != gi)
    w3 = mask.astype(jnp.float32)[:, :, None]
    c1 = c1_ref[...]
    h1n3 = g_ref[...] - c1[:, None, :]
    h1s = x1s_ref[...] - c1
    hw = (h1n3 * w3).reshape(BC * MAX_NB, 128)
    h1n = h1n3.reshape(BC * MAX_NB, 128)
    sh = jnp.sum(hw, axis=0) + jnp.sum(h1s, axis=0)
    sh2 = jnp.sum(hw * h1n, axis=0) + jnp.sum(h1s * h1s, axis=0)
    c = jnp.sum(w3) + jnp.float32(BC)

    @pl.when(pid == 0)
    def _():
        sh_ref[...] = jnp.zeros_like(sh_ref)
        sh2_ref[...] = jnp.zeros_like(sh2_ref)
        cnt_ref[...] = jnp.zeros_like(cnt_ref)

    sh_ref[...] += sh[None, :]
    sh2_ref[...] += sh2[None, :]
    cnt_ref[...] += c


def _stats(G, X1p, C1, nbr, valid):
    return pl.pallas_call(
        _stats_body,
        grid=(M // BC,),
        in_specs=[
            pl.BlockSpec((BC, MAX_NB, 128), lambda i: (i, 0, 0)),
            pl.BlockSpec((BC, 128), lambda i: (i, 0)),
            pl.BlockSpec((BC, 128), lambda i: (i, 0)),
            pl.BlockSpec((BC, MAX_NB), lambda i: (i, 0)),
            pl.BlockSpec((BC, MAX_NB), lambda i: (i, 0)),
        ],
        out_specs=[
            pl.BlockSpec((1, 128), lambda i: (0, 0)),
            pl.BlockSpec((1, 128), lambda i: (0, 0)),
            pl.BlockSpec((1, 1), lambda i: (0, 0)),
        ],
        out_shape=[
            jax.ShapeDtypeStruct((1, 128), jnp.float32),
            jax.ShapeDtypeStruct((1, 128), jnp.float32),
            jax.ShapeDtypeStruct((1, 1), jnp.float32),
        ],
    )(G, X1p, C1, nbr, valid)


def _mlp_body(g_ref, x1s_ref, c1_ref, nbr_ref, val_ref, a_ref, b_ref,
              w2_ref, b2_ref, o_ref):
    pid = pl.program_id(0)
    gi = pid * BC + jax.lax.broadcasted_iota(jnp.int32, (BC, MAX_NB), 0)
    mask = (val_ref[...] != 0) & (nbr_ref[...] != gi)
    c1 = c1_ref[...]
    h1n = (g_ref[...] - c1[:, None, :]).reshape(BC * MAX_NB, 128)
    h1s = x1s_ref[...] - c1
    hall = jnp.concatenate([h1n, h1s], axis=0)
    hall = jnp.maximum(hall * a_ref[...] + b_ref[...], 0.0)
    h2 = jnp.dot(hall, w2_ref[...], preferred_element_type=jnp.float32) + b2_ref[...]
    h2n = h2[: BC * MAX_NB].reshape(BC, MAX_NB, 256)
    h2s = h2[BC * MAX_NB:]
    neg = jnp.float32(-jnp.inf)
    m3 = mask.astype(jnp.float32)[:, :, None]
    h2n = jnp.where(m3 != 0, h2n, neg)
    mx = jnp.max(h2n, axis=1)
    o_ref[...] = jnp.maximum(mx, h2s)


def _mlp(G, X1p, C1, nbr, valid, a, b, W2, b2):
    return pl.pallas_call(
        _mlp_body,
        grid=(M // BC,),
        in_specs=[
            pl.BlockSpec((BC, MAX_NB, 128), lambda i: (i, 0, 0)),
            pl.BlockSpec((BC, 128), lambda i: (i, 0)),
            pl.BlockSpec((BC, 128), lambda i: (i, 0)),
            pl.BlockSpec((BC, MAX_NB), lambda i: (i, 0)),
            pl.BlockSpec((BC, MAX_NB), lambda i: (i, 0)),
            pl.BlockSpec((1, 128), lambda i: (0, 0)),
            pl.BlockSpec((1, 128), lambda i: (0, 0)),
            pl.BlockSpec((128, 256), lambda i: (0, 0)),
            pl.BlockSpec((1, 256), lambda i: (0, 0)),
        ],
        out_specs=pl.BlockSpec((BC, 256), lambda i: (i, 0)),
        out_shape=jax.ShapeDtypeStruct((M, 256), jnp.float32),
    )(G, X1p, C1, nbr, valid, a, b, W2, b2[None, :])


def kernel(x, pos, batch, W1, b1, gamma, beta, W2, b2):
    idx = _fps(pos)
    pos_c = pos[idx]
    X1p = _x1p(x, pos, W1, b1)
    pos_c_pad = jnp.concatenate(
        [pos_c, jnp.broadcast_to(pos_c[:1], (M_PAD - M, 3))], axis=0)
    nbr, valid, G = _sc_select_gather(pos, pos_c_pad, X1p)
    C1 = (pos_c / RADIUS) @ W1[F:]
    sh, sh2, cnt = _stats(G, X1p, C1, nbr, valid)
    cnt = cnt[0, 0]
    mu = sh[0] / cnt
    var = sh2[0] / cnt - mu * mu
    a = gamma / jnp.sqrt(var + 1e-5)
    b = beta - mu * a
    out = _mlp(G, X1p, C1, nbr, valid, a[None, :], b[None, :], W2, b2)
    return (out, pos_c, batch[idx])


# speedup vs baseline: 12.8858x; 1.0887x over previous
import functools

import jax
import jax.numpy as jnp
import numpy as np
from jax import lax
from jax.experimental import pallas as pl
from jax.experimental.pallas import tpu as pltpu
from jax.experimental.pallas import tpu_sc as plsc

N = 10000
F = 128
RATIO = 0.5
RADIUS = 0.2
MAX_NB = 64
M = int(N * RATIO)

NPAD = 10240
ROWS = NPAD // 128


def _fps_body(p0_ref, p1_ref, p2_ref, ps_ref, o_ref, dists_ref):
    BIG = jnp.int32(2**30)
    gidx = (jax.lax.broadcasted_iota(jnp.int32, (ROWS, 128), 0) * 128
            + jax.lax.broadcasted_iota(jnp.int32, (ROWS, 128), 1))
    dists_ref[...] = jnp.where(gidx < N, jnp.inf, -jnp.inf)
    o_ref[0] = 0
    p0 = p0_ref[...]
    p1 = p1_ref[...]
    p2 = p2_ref[...]

    def body(i, carry):
        px, py, pz = carry
        dx = p0 - px
        dy = p1 - py
        dz = p2 - pz
        d = (dx * dx + dz * dz) + dy * dy
        dists = jnp.minimum(dists_ref[...], d)
        dists_ref[...] = dists
        mx = jnp.max(dists)
        last = jnp.min(jnp.where(dists == mx, gidx, BIG))
        o_ref[i] = last
        return (ps_ref[0, last], ps_ref[1, last], ps_ref[2, last])

    jax.lax.fori_loop(1, M, body,
                      (ps_ref[0, 0], ps_ref[1, 0], ps_ref[2, 0]))


def _fps(pos):
    pp = jnp.pad(pos, ((0, NPAD - N), (0, 0))).T.reshape(3, ROWS, 128)
    return pl.pallas_call(
        _fps_body,
        in_specs=[
            pl.BlockSpec((ROWS, 128), lambda: (0, 0)),
            pl.BlockSpec((ROWS, 128), lambda: (0, 0)),
            pl.BlockSpec((ROWS, 128), lambda: (0, 0)),
            pl.BlockSpec(memory_space=pltpu.SMEM),
        ],
        out_specs=pl.BlockSpec(memory_space=pltpu.SMEM),
        out_shape=jax.ShapeDtypeStruct((M,), jnp.int32),
        scratch_shapes=[pltpu.VMEM((ROWS, 128), jnp.float32)],
    )(pp[0], pp[1], pp[2], pos.T)


def _radius(pos, pos_c, r, max_nb):
    d2 = jnp.sum((pos_c[:, None, :] - pos[None, :, :]) ** 2, axis=-1)
    scores = jnp.where(d2 <= r * r, -d2, -jnp.inf)
    vals, nbr = jax.lax.top_k(scores, max_nb)
    valid = vals > -jnp.inf
    return nbr, valid


NSC = 32
WPC = 160
M_PAD = NSC * WPC
R2BITS = 1025758986
NCHUNK = N // 16


def _sc_select_body(px_h, py_h, pz_h, cx_h, cy_h, cz_h, x1p_h,
                    nbr_h, val_h, g_h,
                    px, py, pz, cx, cy, cz,
                    cd2, cidx, nbrs, vals, gbuf0, gbuf1,
                    sem_in, sem_g0, sem_g1, sem_out):
    wid = lax.axis_index("s") * 2 + lax.axis_index("c")
    base_c = wid * WPC
    pltpu.sync_copy(px_h, px)
    pltpu.sync_copy(py_h, py)
    pltpu.sync_copy(pz_h, pz)
    pltpu.sync_copy(cx_h.at[pl.ds(base_c * 16, WPC * 16)], cx)
    pltpu.sync_copy(cy_h.at[pl.ds(base_c * 16, WPC * 16)], cy)
    pltpu.sync_copy(cz_h.at[pl.ds(base_c * 16, WPC * 16)], cz)

    lane = lax.iota(jnp.int32, 16)
    r2 = jnp.float32(RADIUS * RADIUS)

    def per_centroid(c, _):
        ci = base_c + c
        ccx = cx[pl.ds(c * 16, 16)]
        ccy = cy[pl.ds(c * 16, 16)]
        ccz = cz[pl.ds(c * 16, 16)]

        def scan_body(t, off):
            dx = px[pl.ds(t * 16, 16)] - ccx
            dy = py[pl.ds(t * 16, 16)] - ccy
            dz = pz[pl.ds(t * 16, 16)] - ccz
            d2 = (dx * dx + dy * dy) + dz * dz
            m = d2 <= r2
            mi = m.astype(jnp.int32)
            dst = plsc.cumsum(mi) - mi + off
            plsc.store_scatter(cd2, [dst], d2, mask=m)
            plsc.store_scatter(cidx, [dst], lane + t * 16, mask=m)
            return off + jnp.sum(mi)

        cnt_in = lax.fori_loop(0, NCHUNK, scan_body, jnp.int32(0), unroll=5)
        nv = (cnt_in + 15) // 16

        def count_le(v):
            def cb(t, acc):
                d2b = plsc.bitcast(cd2[pl.ds(t * 16, 16)], jnp.int32)
                okm = (lane + t * 16) < cnt_in
                return acc + jnp.sum(((d2b <= v) & okm).astype(jnp.int32))
            return lax.fori_loop(0, nv, cb, jnp.int32(0))

        def bs(_, lohi):
            lo, hi = lohi
            mid = (lo + hi) // 2
            le = count_le(mid)
            big = le >= MAX_NB
            return (jnp.where(big, lo, mid + 1), jnp.where(big, mid, hi))

        lo, hi = lax.fori_loop(0, 31, bs, (jnp.int32(0), jnp.int32(R2BITS)))
        v64 = jnp.where(cnt_in <= MAX_NB, jnp.int32(R2BITS + 1), lo)

        def count_lt(v):
            def cb(t, acc):
                d2b = plsc.bitcast(cd2[pl.ds(t * 16, 16)], jnp.int32)
                okm = (lane + t * 16) < cnt_in
                return acc + jnp.sum(((d2b < v) & okm).astype(jnp.int32))
            return lax.fori_loop(0, nv, cb, jnp.int32(0))

        n_lt = count_lt(v64)
        need = MAX_NB - n_lt

        row = c * MAX_NB
        splat_ci = jnp.full((16,), 0, jnp.int32) + ci
        for q in range(MAX_NB // 16):
            nbrs[pl.ds(row + q * 16, 16)] = splat_ci

        def sel_body(t, carry):
            off2, ties = carry
            d2b = plsc.bitcast(cd2[pl.ds(t * 16, 16)], jnp.int32)
            idxv = cidx[pl.ds(t * 16, 16)]
            okm = (lane + t * 16) < cnt_in
            lt = (d2b < v64) & okm
            eq = (d2b == v64) & okm
            pref = plsc.cumsum(eq.astype(jnp.int32)) + ties
            sel = lt | (eq & (pref <= need))
            seli = sel.astype(jnp.int32)
            dst = plsc.cumsum(seli) - seli + (row + off2)
            plsc.store_scatter(nbrs, [dst], idxv, mask=sel)
            return (off2 + jnp.sum(seli),
                    ties + jnp.sum(eq.astype(jnp.int32)))

        s, _ = lax.fori_loop(0, nv, sel_body, (jnp.int32(0), jnp.int32(0)))

        for q in range(MAX_NB // 16):
            vals[pl.ds(row + q * 16, 16)] = ((lane + q * 16) < s).astype(jnp.int32)
        return _

    lax.fori_loop(0, WPC, per_centroid, 0)

    pltpu.sync_copy(nbrs.at[pl.ds(0, WPC * MAX_NB)],
                    nbr_h.at[pl.ds(base_c * MAX_NB, WPC * MAX_NB)])
    pltpu.sync_copy(vals.at[pl.ds(0, WPC * MAX_NB)],
                    val_h.at[pl.ds(base_c * MAX_NB, WPC * MAX_NB)])

    def gather_one(c, _):
        row = c * MAX_NB
        for q in range(MAX_NB // 16):
            idxv = nbrs[pl.ds(row + q * 16, 16)]
            pltpu.async_copy(x1p_h.at[idxv],
                             gbuf0.at[pl.ds(q * 16, 16)], sem_g0).wait()
        pltpu.sync_copy(gbuf0, g_h.at[pl.ds((base_c + c) * MAX_NB, MAX_NB)])
        return _

    lax.fori_loop(0, WPC, gather_one, 0)


def _sc_select_gather(pos, pos_c_pad, X1p):
    posp = pos.T
    crep = jnp.repeat(pos_c_pad.T.reshape(3, M_PAD), 16, axis=1)
    mesh = plsc.VectorSubcoreMesh(core_axis_name="c", subcore_axis_name="s")
    f = functools.partial(
        pl.kernel,
        mesh=mesh,
        compiler_params=pltpu.CompilerParams(needs_layout_passes=False),
        out_type=[
            jax.ShapeDtypeStruct((M_PAD * MAX_NB,), jnp.int32),
            jax.ShapeDtypeStruct((M_PAD * MAX_NB,), jnp.int32),
            jax.ShapeDtypeStruct((M_PAD * MAX_NB, 128), jnp.float32),
        ],
        scratch_types=[
            pltpu.VMEM((N,), jnp.float32),
            pltpu.VMEM((N,), jnp.float32),
            pltpu.VMEM((N,), jnp.float32),
            pltpu.VMEM((WPC * 16,), jnp.float32),
            pltpu.VMEM((WPC * 16,), jnp.float32),
            pltpu.VMEM((WPC * 16,), jnp.float32),
            pltpu.VMEM((N + 16,), jnp.float32),
            pltpu.VMEM((N + 16,), jnp.int32),
            pltpu.VMEM((WPC * MAX_NB + 16,), jnp.int32),
            pltpu.VMEM((WPC * MAX_NB + 16,), jnp.int32),
            pltpu.VMEM((MAX_NB, 128), jnp.float32),
            pltpu.VMEM((MAX_NB, 128), jnp.float32),
            pltpu.SemaphoreType.DMA,
            pltpu.SemaphoreType.DMA,
            pltpu.SemaphoreType.DMA,
            pltpu.SemaphoreType.DMA,
        ],
    )(_sc_select_body)
    nbr, val, G = f(posp[0], posp[1], posp[2],
                    crep[0], crep[1], crep[2], X1p)
    return (nbr.reshape(M_PAD, MAX_NB), val.reshape(M_PAD, MAX_NB),
            G.reshape(M_PAD, MAX_NB, 128))


def _x1p_body(xin_ref, w_ref, b_ref, o_ref):
    o_ref[...] = (
        jnp.dot(xin_ref[...], w_ref[...], preferred_element_type=jnp.float32)
        + b_ref[...]
    )


def _x1p(x, pos, W1, b1):
    xin = jnp.concatenate([x, pos, jnp.zeros((N, 5), jnp.float32)], axis=1)
    W1p = jnp.concatenate([W1, jnp.zeros((5, 128), jnp.float32)], axis=0)
    BLK = 1000
    return pl.pallas_call(
        _x1p_body,
        grid=(N // BLK,),
        in_specs=[
            pl.BlockSpec((BLK, 136), lambda i: (i, 0)),
            pl.BlockSpec((136, 128), lambda i: (0, 0)),
            pl.BlockSpec((1, 128), lambda i: (0, 0)),
        ],
        out_specs=pl.BlockSpec((BLK, 128), lambda i: (i, 0)),
        out_shape=jax.ShapeDtypeStruct((N, 128), jnp.float32),
    )(xin, W1p, b1[None, :])


BC = 8


def _stats_body(g_ref, x1s_ref, c1_ref, nbr_ref, val_ref, sh_ref, sh2_ref, cnt_ref):
    pid = pl.program_id(0)
    gi = pid * BC + jax.lax.broadcasted_iota(jnp.int32, (BC, MAX_NB), 0)
    mask = (val_ref[...] != 0) & (nbr_ref[...] != gi)
    w3 = mask.astype(jnp.float32)[:, :, None]
    c1 = c1_ref[...]
    h1n3 = g_ref[...] - c1[:, None, :]
    h1s = x1s_ref[...] - c1
    hw = (h1n3 * w3).reshape(BC * MAX_NB, 128)
    h1n = h1n3.reshape(BC * MAX_NB, 128)
    sh = jnp.sum(hw, axis=0) + jnp.sum(h1s, axis=0)
    sh2 = jnp.sum(hw * h1n, axis=0) + jnp.sum(h1s * h1s, axis=0)
    c = jnp.sum(w3) + jnp.float32(BC)

    @pl.when(pid == 0)
    def _():
        sh_ref[...] = jnp.zeros_like(sh_ref)
        sh2_ref[...] = jnp.zeros_like(sh2_ref)
        cnt_ref[...] = jnp.zeros_like(cnt_ref)

    sh_ref[...] += sh[None, :]
    sh2_ref[...] += sh2[None, :]
    cnt_ref[...] += c


def _stats(G, X1p, C1, nbr, valid):
    return pl.pallas_call(
        _stats_body,
        grid=(M // BC,),
        in_specs=[
            pl.BlockSpec((BC, MAX_NB, 128), lambda i: (i, 0, 0)),
            pl.BlockSpec((BC, 128), lambda i: (i, 0)),
            pl.BlockSpec((BC, 128), lambda i: (i, 0)),
            pl.BlockSpec((BC, MAX_NB), lambda i: (i, 0)),
            pl.BlockSpec((BC, MAX_NB), lambda i: (i, 0)),
        ],
        out_specs=[
            pl.BlockSpec((1, 128), lambda i: (0, 0)),
            pl.BlockSpec((1, 128), lambda i: (0, 0)),
            pl.BlockSpec((1, 1), lambda i: (0, 0)),
        ],
        out_shape=[
            jax.ShapeDtypeStruct((1, 128), jnp.float32),
            jax.ShapeDtypeStruct((1, 128), jnp.float32),
            jax.ShapeDtypeStruct((1, 1), jnp.float32),
        ],
    )(G, X1p, C1, nbr, valid)


def _mlp_body(g_ref, x1s_ref, c1_ref, nbr_ref, val_ref, a_ref, b_ref,
              w2_ref, b2_ref, o_ref):
    pid = pl.program_id(0)
    gi = pid * BC + jax.lax.broadcasted_iota(jnp.int32, (BC, MAX_NB), 0)
    mask = (val_ref[...] != 0) & (nbr_ref[...] != gi)
    c1 = c1_ref[...]
    h1n = (g_ref[...] - c1[:, None, :]).reshape(BC * MAX_NB, 128)
    h1s = x1s_ref[...] - c1
    hall = jnp.concatenate([h1n, h1s], axis=0)
    hall = jnp.maximum(hall * a_ref[...] + b_ref[...], 0.0)
    h2 = jnp.dot(hall, w2_ref[...], preferred_element_type=jnp.float32) + b2_ref[...]
    h2n = h2[: BC * MAX_NB].reshape(BC, MAX_NB, 256)
    h2s = h2[BC * MAX_NB:]
    neg = jnp.float32(-jnp.inf)
    m3 = mask.astype(jnp.float32)[:, :, None]
    h2n = jnp.where(m3 != 0, h2n, neg)
    mx = jnp.max(h2n, axis=1)
    o_ref[...] = jnp.maximum(mx, h2s)


def _mlp(G, X1p, C1, nbr, valid, a, b, W2, b2):
    return pl.pallas_call(
        _mlp_body,
        grid=(M // BC,),
        in_specs=[
            pl.BlockSpec((BC, MAX_NB, 128), lambda i: (i, 0, 0)),
            pl.BlockSpec((BC, 128), lambda i: (i, 0)),
            pl.BlockSpec((BC, 128), lambda i: (i, 0)),
            pl.BlockSpec((BC, MAX_NB), lambda i: (i, 0)),
            pl.BlockSpec((BC, MAX_NB), lambda i: (i, 0)),
            pl.BlockSpec((1, 128), lambda i: (0, 0)),
            pl.BlockSpec((1, 128), lambda i: (0, 0)),
            pl.BlockSpec((128, 256), lambda i: (0, 0)),
            pl.BlockSpec((1, 256), lambda i: (0, 0)),
        ],
        out_specs=pl.BlockSpec((BC, 256), lambda i: (i, 0)),
        out_shape=jax.ShapeDtypeStruct((M, 256), jnp.float32),
    )(G, X1p, C1, nbr, valid, a, b, W2, b2[None, :])


def kernel(x, pos, batch, W1, b1, gamma, beta, W2, b2):
    idx = _fps(pos)
    pos_c = pos[idx]
    X1p = _x1p(x, pos, W1, b1)
    pos_c_pad = jnp.concatenate(
        [pos_c, jnp.broadcast_to(pos_c[:1], (M_PAD - M, 3))], axis=0)
    nbr, valid, G = _sc_select_gather(pos, pos_c_pad, X1p)
    C1 = (pos_c / RADIUS) @ W1[F:]
    sh, sh2, cnt = _stats(G, X1p, C1, nbr, valid)
    cnt = cnt[0, 0]
    mu = sh[0] / cnt
    var = sh2[0] / cnt - mu * mu
    a = gamma / jnp.sqrt(var + 1e-5)
    b = beta - mu * a
    out = _mlp(G, X1p, C1, nbr, valid, a[None, :], b[None, :], W2, b2)
    return (out, pos_c, batch[idx])


# speedup vs baseline: 14.3237x; 1.1116x over previous
import functools

import jax
import jax.numpy as jnp
import numpy as np
from jax import lax
from jax.experimental import pallas as pl
from jax.experimental.pallas import tpu as pltpu
from jax.experimental.pallas import tpu_sc as plsc

N = 10000
F = 128
RATIO = 0.5
RADIUS = 0.2
MAX_NB = 64
M = int(N * RATIO)

NPAD = 10240
ROWS = NPAD // 128


def _fps_body(p0_ref, p1_ref, p2_ref, ps_ref, o_ref, dists_ref):
    BIG = jnp.int32(2**30)
    gidx = (jax.lax.broadcasted_iota(jnp.int32, (ROWS, 128), 0) * 128
            + jax.lax.broadcasted_iota(jnp.int32, (ROWS, 128), 1))
    dists_ref[...] = jnp.where(gidx < N, jnp.inf, -jnp.inf)
    o_ref[0] = 0
    p0 = p0_ref[...]
    p1 = p1_ref[...]
    p2 = p2_ref[...]

    def body(i, carry):
        px, py, pz = carry
        dx = p0 - px
        dy = p1 - py
        dz = p2 - pz
        d = (dx * dx + dz * dz) + dy * dy
        dists = jnp.minimum(dists_ref[...], d)
        dists_ref[...] = dists
        mx = jnp.max(dists)
        last = jnp.min(jnp.where(dists == mx, gidx, BIG))
        o_ref[i] = last
        return (ps_ref[0, last], ps_ref[1, last], ps_ref[2, last])

    jax.lax.fori_loop(1, M, body,
                      (ps_ref[0, 0], ps_ref[1, 0], ps_ref[2, 0]))


def _fps(pos):
    pp = jnp.pad(pos, ((0, NPAD - N), (0, 0))).T.reshape(3, ROWS, 128)
    return pl.pallas_call(
        _fps_body,
        in_specs=[
            pl.BlockSpec((ROWS, 128), lambda: (0, 0)),
            pl.BlockSpec((ROWS, 128), lambda: (0, 0)),
            pl.BlockSpec((ROWS, 128), lambda: (0, 0)),
            pl.BlockSpec(memory_space=pltpu.SMEM),
        ],
        out_specs=pl.BlockSpec(memory_space=pltpu.SMEM),
        out_shape=jax.ShapeDtypeStruct((M,), jnp.int32),
        scratch_shapes=[pltpu.VMEM((ROWS, 128), jnp.float32)],
    )(pp[0], pp[1], pp[2], pos.T)


def _radius(pos, pos_c, r, max_nb):
    d2 = jnp.sum((pos_c[:, None, :] - pos[None, :, :]) ** 2, axis=-1)
    scores = jnp.where(d2 <= r * r, -d2, -jnp.inf)
    vals, nbr = jax.lax.top_k(scores, max_nb)
    valid = vals > -jnp.inf
    return nbr, valid


NSC = 32
WPC = 160
M_PAD = NSC * WPC
R2BITS = 1025758986
NCHUNK = N // 16


def _sc_select_body(px_h, py_h, pz_h, cx_h, cy_h, cz_h, x1p_h,
                    nbr_h, val_h, g_h,
                    px, py, pz, cx, cy, cz,
                    cd2, cidx, nbrs, vals, gbuf0, gbuf1,
                    sem_in, sem_g0, sem_g1, sem_out):
    wid = lax.axis_index("s") * 2 + lax.axis_index("c")
    base_c = wid * WPC
    pltpu.sync_copy(px_h, px)
    pltpu.sync_copy(py_h, py)
    pltpu.sync_copy(pz_h, pz)
    pltpu.sync_copy(cx_h.at[pl.ds(base_c * 16, WPC * 16)], cx)
    pltpu.sync_copy(cy_h.at[pl.ds(base_c * 16, WPC * 16)], cy)
    pltpu.sync_copy(cz_h.at[pl.ds(base_c * 16, WPC * 16)], cz)

    lane = lax.iota(jnp.int32, 16)
    r2 = jnp.float32(RADIUS * RADIUS)

    def per_centroid(c, _):
        ci = base_c + c
        ccx = cx[pl.ds(c * 16, 16)]
        ccy = cy[pl.ds(c * 16, 16)]
        ccz = cz[pl.ds(c * 16, 16)]

        zero16 = jnp.full((16,), 0, jnp.int32)

        def scan_body(t, off_vec):
            dx = px[pl.ds(t * 16, 16)] - ccx
            dy = py[pl.ds(t * 16, 16)] - ccy
            dz = pz[pl.ds(t * 16, 16)] - ccz
            d2 = (dx * dx + dy * dy) + dz * dz
            m = d2 <= r2
            mi = m.astype(jnp.int32)
            dst = plsc.cumsum(mi) - mi + off_vec
            plsc.store_scatter(cd2, [dst], d2, mask=m)
            plsc.store_scatter(cidx, [dst], lane + t * 16, mask=m)
            return off_vec + plsc.all_reduce_population_count(m)

        off_vec = lax.fori_loop(0, NCHUNK, scan_body, zero16, unroll=5)
        cnt_in = jnp.sum(off_vec) // 16
        nv = (cnt_in + 15) // 16

        def count_le(v):
            def cb(t, acc):
                d2b = plsc.bitcast(cd2[pl.ds(t * 16, 16)], jnp.int32)
                okm = (lane + t * 16) < cnt_in
                return acc + plsc.all_reduce_population_count((d2b <= v) & okm)
            acc = lax.fori_loop(0, nv, cb, zero16)
            return jnp.sum(acc) // 16

        def bs(_, lohi):
            lo, hi = lohi
            mid = (lo + hi) // 2
            le = count_le(mid)
            big = le >= MAX_NB
            return (jnp.where(big, lo, mid + 1), jnp.where(big, mid, hi))

        lo, hi = lax.fori_loop(0, 31, bs, (jnp.int32(0), jnp.int32(R2BITS)))
        v64 = jnp.where(cnt_in <= MAX_NB, jnp.int32(R2BITS + 1), lo)

        def count_lt(v):
            def cb(t, acc):
                d2b = plsc.bitcast(cd2[pl.ds(t * 16, 16)], jnp.int32)
                okm = (lane + t * 16) < cnt_in
                return acc + plsc.all_reduce_population_count((d2b < v) & okm)
            acc = lax.fori_loop(0, nv, cb, zero16)
            return jnp.sum(acc) // 16

        n_lt = count_lt(v64)
        need = MAX_NB - n_lt

        row = c * MAX_NB
        splat_ci = jnp.full((16,), 0, jnp.int32) + ci
        for q in range(MAX_NB // 16):
            nbrs[pl.ds(row + q * 16, 16)] = splat_ci

        def sel_body(t, carry):
            off2, ties = carry
            d2b = plsc.bitcast(cd2[pl.ds(t * 16, 16)], jnp.int32)
            idxv = cidx[pl.ds(t * 16, 16)]
            okm = (lane + t * 16) < cnt_in
            lt = (d2b < v64) & okm
            eq = (d2b == v64) & okm
            pref = plsc.cumsum(eq.astype(jnp.int32)) + ties
            sel = lt | (eq & (pref <= need))
            seli = sel.astype(jnp.int32)
            dst = plsc.cumsum(seli) - seli + (row + off2)
            plsc.store_scatter(nbrs, [dst], idxv, mask=sel)
            return (off2 + jnp.sum(seli),
                    ties + jnp.sum(eq.astype(jnp.int32)))

        s, _ = lax.fori_loop(0, nv, sel_body, (jnp.int32(0), jnp.int32(0)))

        for q in range(MAX_NB // 16):
            vals[pl.ds(row + q * 16, 16)] = ((lane + q * 16) < s).astype(jnp.int32)
        return _

    lax.fori_loop(0, WPC, per_centroid, 0)

    pltpu.sync_copy(nbrs.at[pl.ds(0, WPC * MAX_NB)],
                    nbr_h.at[pl.ds(base_c * MAX_NB, WPC * MAX_NB)])
    pltpu.sync_copy(vals.at[pl.ds(0, WPC * MAX_NB)],
                    val_h.at[pl.ds(base_c * MAX_NB, WPC * MAX_NB)])

    def gather_pair(p, _):
        c0 = 2 * p
        c1 = 2 * p + 1
        handles = []
        for (c, buf, sem) in ((c0, gbuf0, sem_g0), (c1, gbuf1, sem_g1)):
            row = c * MAX_NB
            for q in range(MAX_NB // 16):
                idxv = nbrs[pl.ds(row + q * 16, 16)]
                handles.append(pltpu.async_copy(
                    x1p_h.at[idxv], buf.at[pl.ds(q * 16, 16)], sem))
        for h in handles[:4]:
            h.wait()
        pltpu.sync_copy(gbuf0, g_h.at[pl.ds((base_c + c0) * MAX_NB, MAX_NB)])
        for h in handles[4:]:
            h.wait()
        pltpu.sync_copy(gbuf1, g_h.at[pl.ds((base_c + c1) * MAX_NB, MAX_NB)])
        return _

    lax.fori_loop(0, WPC // 2, gather_pair, 0)


def _sc_select_gather(pos, pos_c_pad, X1p):
    posp = pos.T
    crep = jnp.repeat(pos_c_pad.T.reshape(3, M_PAD), 16, axis=1)
    mesh = plsc.VectorSubcoreMesh(core_axis_name="c", subcore_axis_name="s")
    f = functools.partial(
        pl.kernel,
        mesh=mesh,
        compiler_params=pltpu.CompilerParams(needs_layout_passes=False),
        out_type=[
            jax.ShapeDtypeStruct((M_PAD * MAX_NB,), jnp.int32),
            jax.ShapeDtypeStruct((M_PAD * MAX_NB,), jnp.int32),
            jax.ShapeDtypeStruct((M_PAD * MAX_NB, 128), jnp.float32),
        ],
        scratch_types=[
            pltpu.VMEM((N,), jnp.float32),
            pltpu.VMEM((N,), jnp.float32),
            pltpu.VMEM((N,), jnp.float32),
            pltpu.VMEM((WPC * 16,), jnp.float32),
            pltpu.VMEM((WPC * 16,), jnp.float32),
            pltpu.VMEM((WPC * 16,), jnp.float32),
            pltpu.VMEM((N + 16,), jnp.float32),
            pltpu.VMEM((N + 16,), jnp.int32),
            pltpu.VMEM((WPC * MAX_NB + 16,), jnp.int32),
            pltpu.VMEM((WPC * MAX_NB + 16,), jnp.int32),
            pltpu.VMEM((MAX_NB, 128), jnp.float32),
            pltpu.VMEM((MAX_NB, 128), jnp.float32),
            pltpu.SemaphoreType.DMA,
            pltpu.SemaphoreType.DMA,
            pltpu.SemaphoreType.DMA,
            pltpu.SemaphoreType.DMA,
        ],
    )(_sc_select_body)
    nbr, val, G = f(posp[0], posp[1], posp[2],
                    crep[0], crep[1], crep[2], X1p)
    return (nbr.reshape(M_PAD, MAX_NB), val.reshape(M_PAD, MAX_NB),
            G.reshape(M_PAD, MAX_NB, 128))


def _x1p_body(xin_ref, w_ref, b_ref, o_ref):
    o_ref[...] = (
        jnp.dot(xin_ref[...], w_ref[...], preferred_element_type=jnp.float32)
        + b_ref[...]
    )


def _x1p(x, pos, W1, b1):
    xin = jnp.concatenate([x, pos, jnp.zeros((N, 5), jnp.float32)], axis=1)
    W1p = jnp.concatenate([W1, jnp.zeros((5, 128), jnp.float32)], axis=0)
    BLK = 1000
    return pl.pallas_call(
        _x1p_body,
        grid=(N // BLK,),
        in_specs=[
            pl.BlockSpec((BLK, 136), lambda i: (i, 0)),
            pl.BlockSpec((136, 128), lambda i: (0, 0)),
            pl.BlockSpec((1, 128), lambda i: (0, 0)),
        ],
        out_specs=pl.BlockSpec((BLK, 128), lambda i: (i, 0)),
        out_shape=jax.ShapeDtypeStruct((N, 128), jnp.float32),
    )(xin, W1p, b1[None, :])


BC = 8


def _stats_body(g_ref, x1s_ref, c1_ref, nbr_ref, val_ref, sh_ref, sh2_ref, cnt_ref):
    pid = pl.program_id(0)
    gi = pid * BC + jax.lax.broadcasted_iota(jnp.int32, (BC, MAX_NB), 0)
    mask = (val_ref[...] != 0) & (nbr_ref[...] != gi)
    w3 = mask.astype(jnp.float32)[:, :, None]
    c1 = c1_ref[...]
    h1n3 = g_ref[...] - c1[:, None, :]
    h1s = x1s_ref[...] - c1
    hw = (h1n3 * w3).reshape(BC * MAX_NB, 128)
    h1n = h1n3.reshape(BC * MAX_NB, 128)
    sh = jnp.sum(hw, axis=0) + jnp.sum(h1s, axis=0)
    sh2 = jnp.sum(hw * h1n, axis=0) + jnp.sum(h1s * h1s, axis=0)
    c = jnp.sum(w3) + jnp.float32(BC)

    @pl.when(pid == 0)
    def _():
        sh_ref[...] = jnp.zeros_like(sh_ref)
        sh2_ref[...] = jnp.zeros_like(sh2_ref)
        cnt_ref[...] = jnp.zeros_like(cnt_ref)

    sh_ref[...] += sh[None, :]
    sh2_ref[...] += sh2[None, :]
    cnt_ref[...] += c


def _stats(G, X1p, C1, nbr, valid):
    return pl.pallas_call(
        _stats_body,
        grid=(M // BC,),
        in_specs=[
            pl.BlockSpec((BC, MAX_NB, 128), lambda i: (i, 0, 0)),
            pl.BlockSpec((BC, 128), lambda i: (i, 0)),
            pl.BlockSpec((BC, 128), lambda i: (i, 0)),
            pl.BlockSpec((BC, MAX_NB), lambda i: (i, 0)),
            pl.BlockSpec((BC, MAX_NB), lambda i: (i, 0)),
        ],
        out_specs=[
            pl.BlockSpec((1, 128), lambda i: (0, 0)),
            pl.BlockSpec((1, 128), lambda i: (0, 0)),
            pl.BlockSpec((1, 1), lambda i: (0, 0)),
        ],
        out_shape=[
            jax.ShapeDtypeStruct((1, 128), jnp.float32),
            jax.ShapeDtypeStruct((1, 128), jnp.float32),
            jax.ShapeDtypeStruct((1, 1), jnp.float32),
        ],
    )(G, X1p, C1, nbr, valid)


def _mlp_body(g_ref, x1s_ref, c1_ref, nbr_ref, val_ref, a_ref, b_ref,
              w2_ref, b2_ref, o_ref):
    pid = pl.program_id(0)
    gi = pid * BC + jax.lax.broadcasted_iota(jnp.int32, (BC, MAX_NB), 0)
    mask = (val_ref[...] != 0) & (nbr_ref[...] != gi)
    c1 = c1_ref[...]
    h1n = (g_ref[...] - c1[:, None, :]).reshape(BC * MAX_NB, 128)
    h1s = x1s_ref[...] - c1
    hall = jnp.concatenate([h1n, h1s], axis=0)
    hall = jnp.maximum(hall * a_ref[...] + b_ref[...], 0.0)
    h2 = jnp.dot(hall, w2_ref[...], preferred_element_type=jnp.float32) + b2_ref[...]
    h2n = h2[: BC * MAX_NB].reshape(BC, MAX_NB, 256)
    h2s = h2[BC * MAX_NB:]
    neg = jnp.float32(-jnp.inf)
    m3 = mask.astype(jnp.float32)[:, :, None]
    h2n = jnp.where(m3 != 0, h2n, neg)
    mx = jnp.max(h2n, axis=1)
    o_ref[...] = jnp.maximum(mx, h2s)


def _mlp(G, X1p, C1, nbr, valid, a, b, W2, b2):
    return pl.pallas_call(
        _mlp_body,
        grid=(M // BC,),
        in_specs=[
            pl.BlockSpec((BC, MAX_NB, 128), lambda i: (i, 0, 0)),
            pl.BlockSpec((BC, 128), lambda i: (i, 0)),
            pl.BlockSpec((BC, 128), lambda i: (i, 0)),
            pl.BlockSpec((BC, MAX_NB), lambda i: (i, 0)),
            pl.BlockSpec((BC, MAX_NB), lambda i: (i, 0)),
            pl.BlockSpec((1, 128), lambda i: (0, 0)),
            pl.BlockSpec((1, 128), lambda i: (0, 0)),
            pl.BlockSpec((128, 256), lambda i: (0, 0)),
            pl.BlockSpec((1, 256), lambda i: (0, 0)),
        ],
        out_specs=pl.BlockSpec((BC, 256), lambda i: (i, 0)),
        out_shape=jax.ShapeDtypeStruct((M, 256), jnp.float32),
    )(G, X1p, C1, nbr, valid, a, b, W2, b2[None, :])


def kernel(x, pos, batch, W1, b1, gamma, beta, W2, b2):
    idx = _fps(pos)
    pos_c = pos[idx]
    X1p = _x1p(x, pos, W1, b1)
    pos_c_pad = jnp.concatenate(
        [pos_c, jnp.broadcast_to(pos_c[:1], (M_PAD - M, 3))], axis=0)
    nbr, valid, G = _sc_select_gather(pos, pos_c_pad, X1p)
    C1 = (pos_c / RADIUS) @ W1[F:]
    sh, sh2, cnt = _stats(G, X1p, C1, nbr, valid)
    cnt = cnt[0, 0]
    mu = sh[0] / cnt
    var = sh2[0] / cnt - mu * mu
    a = gamma / jnp.sqrt(var + 1e-5)
    b = beta - mu * a
    out = _mlp(G, X1p, C1, nbr, valid, a[None, :], b[None, :], W2, b2)
    return (out, pos_c, batch[idx])


# speedup vs baseline: 14.4576x; 1.0093x over previous
import functools

import jax
import jax.numpy as jnp
import numpy as np
from jax import lax
from jax.experimental import pallas as pl
from jax.experimental.pallas import tpu as pltpu
from jax.experimental.pallas import tpu_sc as plsc

N = 10000
F = 128
RATIO = 0.5
RADIUS = 0.2
MAX_NB = 64
M = int(N * RATIO)

NPAD = 10240
ROWS = NPAD // 128


def _fps_body(p0_ref, p1_ref, p2_ref, ps_ref, o_ref, dists_ref):
    BIG = jnp.int32(2**30)
    gidx = (jax.lax.broadcasted_iota(jnp.int32, (ROWS, 128), 0) * 128
            + jax.lax.broadcasted_iota(jnp.int32, (ROWS, 128), 1))
    dists_ref[...] = jnp.where(gidx < N, jnp.inf, -jnp.inf)
    o_ref[0] = 0
    p0 = p0_ref[...]
    p1 = p1_ref[...]
    p2 = p2_ref[...]

    def body(i, carry):
        px, py, pz = carry
        dx = p0 - px
        dy = p1 - py
        dz = p2 - pz
        d = (dx * dx + dz * dz) + dy * dy
        dists = jnp.minimum(dists_ref[...], d)
        dists_ref[...] = dists
        mx = jnp.max(dists)
        last = jnp.min(jnp.where(dists == mx, gidx, BIG))
        o_ref[i] = last
        return (ps_ref[0, last], ps_ref[1, last], ps_ref[2, last])

    jax.lax.fori_loop(1, M, body,
                      (ps_ref[0, 0], ps_ref[1, 0], ps_ref[2, 0]))


def _fps(pos):
    pp = jnp.pad(pos, ((0, NPAD - N), (0, 0))).T.reshape(3, ROWS, 128)
    return pl.pallas_call(
        _fps_body,
        in_specs=[
            pl.BlockSpec((ROWS, 128), lambda: (0, 0)),
            pl.BlockSpec((ROWS, 128), lambda: (0, 0)),
            pl.BlockSpec((ROWS, 128), lambda: (0, 0)),
            pl.BlockSpec(memory_space=pltpu.SMEM),
        ],
        out_specs=pl.BlockSpec(memory_space=pltpu.SMEM),
        out_shape=jax.ShapeDtypeStruct((M,), jnp.int32),
        scratch_shapes=[pltpu.VMEM((ROWS, 128), jnp.float32)],
    )(pp[0], pp[1], pp[2], pos.T)


def _radius(pos, pos_c, r, max_nb):
    d2 = jnp.sum((pos_c[:, None, :] - pos[None, :, :]) ** 2, axis=-1)
    scores = jnp.where(d2 <= r * r, -d2, -jnp.inf)
    vals, nbr = jax.lax.top_k(scores, max_nb)
    valid = vals > -jnp.inf
    return nbr, valid


NSC = 32
WPC = 160
M_PAD = NSC * WPC
R2BITS = 1025758986
NCHUNK = N // 16


def _sc_select_body(px_h, py_h, pz_h, cx_h, cy_h, cz_h, x1p_h,
                    nbr_h, val_h, g_h,
                    px, py, pz, cx, cy, cz,
                    cd2, cidx, nbrs, vals, gbuf0, gbuf1,
                    sem_in, sem_g0, sem_g1, sem_out):
    wid = lax.axis_index("s") * 2 + lax.axis_index("c")
    base_c = wid * WPC
    pltpu.sync_copy(px_h, px)
    pltpu.sync_copy(py_h, py)
    pltpu.sync_copy(pz_h, pz)
    pltpu.sync_copy(cx_h.at[pl.ds(base_c * 16, WPC * 16)], cx)
    pltpu.sync_copy(cy_h.at[pl.ds(base_c * 16, WPC * 16)], cy)
    pltpu.sync_copy(cz_h.at[pl.ds(base_c * 16, WPC * 16)], cz)

    lane = lax.iota(jnp.int32, 16)
    r2 = jnp.float32(RADIUS * RADIUS)

    def per_centroid(c, _):
        ci = base_c + c
        ccx = cx[pl.ds(c * 16, 16)]
        ccy = cy[pl.ds(c * 16, 16)]
        ccz = cz[pl.ds(c * 16, 16)]

        zero16 = jnp.full((16,), 0, jnp.int32)

        def scan_body(t, off_vec):
            dx = px[pl.ds(t * 16, 16)] - ccx
            dy = py[pl.ds(t * 16, 16)] - ccy
            dz = pz[pl.ds(t * 16, 16)] - ccz
            d2 = (dx * dx + dy * dy) + dz * dz
            m = d2 <= r2
            mi = m.astype(jnp.int32)
            dst = plsc.cumsum(mi) - mi + off_vec
            plsc.store_scatter(cd2, [dst], d2, mask=m)
            plsc.store_scatter(cidx, [dst], lane + t * 16, mask=m)
            return off_vec + plsc.all_reduce_population_count(m)

        off_vec = lax.fori_loop(0, NCHUNK, scan_body, zero16, unroll=5)
        cnt_in = jnp.sum(off_vec) // 16
        nv = (cnt_in + 15) // 16

        def count_le(v):
            def cb(t, acc):
                d2b = plsc.bitcast(cd2[pl.ds(t * 16, 16)], jnp.int32)
                okm = (lane + t * 16) < cnt_in
                return acc + plsc.all_reduce_population_count((d2b <= v) & okm)
            acc = lax.fori_loop(0, nv, cb, zero16)
            return jnp.sum(acc) // 16

        def bs(_, lohi):
            lo, hi = lohi
            mid = (lo + hi) // 2
            le = count_le(mid)
            big = le >= MAX_NB
            return (jnp.where(big, lo, mid + 1), jnp.where(big, mid, hi))

        lo, hi = lax.fori_loop(0, 31, bs, (jnp.int32(0), jnp.int32(R2BITS)))
        v64 = jnp.where(cnt_in <= MAX_NB, jnp.int32(R2BITS + 1), lo)

        def count_lt(v):
            def cb(t, acc):
                d2b = plsc.bitcast(cd2[pl.ds(t * 16, 16)], jnp.int32)
                okm = (lane + t * 16) < cnt_in
                return acc + plsc.all_reduce_population_count((d2b < v) & okm)
            acc = lax.fori_loop(0, nv, cb, zero16)
            return jnp.sum(acc) // 16

        n_lt = count_lt(v64)
        need = MAX_NB - n_lt

        row = c * MAX_NB
        splat_ci = jnp.full((16,), 0, jnp.int32) + ci
        for q in range(MAX_NB // 16):
            nbrs[pl.ds(row + q * 16, 16)] = splat_ci

        def sel_body(t, carry):
            off2, ties = carry
            d2b = plsc.bitcast(cd2[pl.ds(t * 16, 16)], jnp.int32)
            idxv = cidx[pl.ds(t * 16, 16)]
            okm = (lane + t * 16) < cnt_in
            lt = (d2b < v64) & okm
            eq = (d2b == v64) & okm
            pref = plsc.cumsum(eq.astype(jnp.int32)) + ties
            sel = lt | (eq & (pref <= need))
            seli = sel.astype(jnp.int32)
            dst = plsc.cumsum(seli) - seli + (row + off2)
            plsc.store_scatter(nbrs, [dst], idxv, mask=sel)
            return (off2 + jnp.sum(seli),
                    ties + jnp.sum(eq.astype(jnp.int32)))

        s, _ = lax.fori_loop(0, nv, sel_body, (jnp.int32(0), jnp.int32(0)))

        for q in range(MAX_NB // 16):
            vals[pl.ds(row + q * 16, 16)] = ((lane + q * 16) < s).astype(jnp.int32)
        return _

    lax.fori_loop(0, WPC, per_centroid, 0)

    pltpu.sync_copy(nbrs.at[pl.ds(0, WPC * MAX_NB)],
                    nbr_h.at[pl.ds(base_c * MAX_NB, WPC * MAX_NB)])
    pltpu.sync_copy(vals.at[pl.ds(0, WPC * MAX_NB)],
                    val_h.at[pl.ds(base_c * MAX_NB, WPC * MAX_NB)])

    def gather_pair(p, _):
        c0 = 2 * p
        c1 = 2 * p + 1
        h0 = pltpu.async_copy(
            x1p_h.at[nbrs.at[pl.ds(c0 * MAX_NB, MAX_NB)]], gbuf0, sem_g0)
        h1 = pltpu.async_copy(
            x1p_h.at[nbrs.at[pl.ds(c1 * MAX_NB, MAX_NB)]], gbuf1, sem_g1)
        h0.wait()
        pltpu.sync_copy(gbuf0, g_h.at[pl.ds((base_c + c0) * MAX_NB, MAX_NB)])
        h1.wait()
        pltpu.sync_copy(gbuf1, g_h.at[pl.ds((base_c + c1) * MAX_NB, MAX_NB)])
        return _

    lax.fori_loop(0, WPC // 2, gather_pair, 0)


def _sc_select_gather(pos, pos_c_pad, X1p):
    posp = pos.T
    crep = jnp.repeat(pos_c_pad.T.reshape(3, M_PAD), 16, axis=1)
    mesh = plsc.VectorSubcoreMesh(core_axis_name="c", subcore_axis_name="s")
    f = functools.partial(
        pl.kernel,
        mesh=mesh,
        compiler_params=pltpu.CompilerParams(needs_layout_passes=False),
        out_type=[
            jax.ShapeDtypeStruct((M_PAD * MAX_NB,), jnp.int32),
            jax.ShapeDtypeStruct((M_PAD * MAX_NB,), jnp.int32),
            jax.ShapeDtypeStruct((M_PAD * MAX_NB, 128), jnp.float32),
        ],
        scratch_types=[
            pltpu.VMEM((N,), jnp.float32),
            pltpu.VMEM((N,), jnp.float32),
            pltpu.VMEM((N,), jnp.float32),
            pltpu.VMEM((WPC * 16,), jnp.float32),
            pltpu.VMEM((WPC * 16,), jnp.float32),
            pltpu.VMEM((WPC * 16,), jnp.float32),
            pltpu.VMEM((N + 16,), jnp.float32),
            pltpu.VMEM((N + 16,), jnp.int32),
            pltpu.VMEM((WPC * MAX_NB + 16,), jnp.int32),
            pltpu.VMEM((WPC * MAX_NB + 16,), jnp.int32),
            pltpu.VMEM((MAX_NB, 128), jnp.float32),
            pltpu.VMEM((MAX_NB, 128), jnp.float32),
            pltpu.SemaphoreType.DMA,
            pltpu.SemaphoreType.DMA,
            pltpu.SemaphoreType.DMA,
            pltpu.SemaphoreType.DMA,
        ],
    )(_sc_select_body)
    nbr, val, G = f(posp[0], posp[1], posp[2],
                    crep[0], crep[1], crep[2], X1p)
    return (nbr.reshape(M_PAD, MAX_NB), val.reshape(M_PAD, MAX_NB),
            G.reshape(M_PAD, MAX_NB, 128))


def _x1p_body(xin_ref, w_ref, b_ref, o_ref):
    o_ref[...] = (
        jnp.dot(xin_ref[...], w_ref[...], preferred_element_type=jnp.float32)
        + b_ref[...]
    )


def _x1p(x, pos, W1, b1):
    xin = jnp.concatenate([x, pos, jnp.zeros((N, 5), jnp.float32)], axis=1)
    W1p = jnp.concatenate([W1, jnp.zeros((5, 128), jnp.float32)], axis=0)
    BLK = 1000
    return pl.pallas_call(
        _x1p_body,
        grid=(N // BLK,),
        in_specs=[
            pl.BlockSpec((BLK, 136), lambda i: (i, 0)),
            pl.BlockSpec((136, 128), lambda i: (0, 0)),
            pl.BlockSpec((1, 128), lambda i: (0, 0)),
        ],
        out_specs=pl.BlockSpec((BLK, 128), lambda i: (i, 0)),
        out_shape=jax.ShapeDtypeStruct((N, 128), jnp.float32),
    )(xin, W1p, b1[None, :])


BC = 8


def _stats_body(g_ref, x1s_ref, c1_ref, nbr_ref, val_ref, sh_ref, sh2_ref, cnt_ref):
    pid = pl.program_id(0)
    gi = pid * BC + jax.lax.broadcasted_iota(jnp.int32, (BC, MAX_NB), 0)
    mask = (val_ref[...] != 0) & (nbr_ref[...] != gi)
    w3 = mask.astype(jnp.float32)[:, :, None]
    c1 = c1_ref[...]
    h1n3 = g_ref[...] - c1[:, None, :]
    h1s = x1s_ref[...] - c1
    hw = (h1n3 * w3).reshape(BC * MAX_NB, 128)
    h1n = h1n3.reshape(BC * MAX_NB, 128)
    sh = jnp.sum(hw, axis=0) + jnp.sum(h1s, axis=0)
    sh2 = jnp.sum(hw * h1n, axis=0) + jnp.sum(h1s * h1s, axis=0)
    c = jnp.sum(w3) + jnp.float32(BC)

    @pl.when(pid == 0)
    def _():
        sh_ref[...] = jnp.zeros_like(sh_ref)
        sh2_ref[...] = jnp.zeros_like(sh2_ref)
        cnt_ref[...] = jnp.zeros_like(cnt_ref)

    sh_ref[...] += sh[None, :]
    sh2_ref[...] += sh2[None, :]
    cnt_ref[...] += c


def _stats(G, X1p, C1, nbr, valid):
    return pl.pallas_call(
        _stats_body,
        grid=(M // BC,),
        in_specs=[
            pl.BlockSpec((BC, MAX_NB, 128), lambda i: (i, 0, 0)),
            pl.BlockSpec((BC, 128), lambda i: (i, 0)),
            pl.BlockSpec((BC, 128), lambda i: (i, 0)),
            pl.BlockSpec((BC, MAX_NB), lambda i: (i, 0)),
            pl.BlockSpec((BC, MAX_NB), lambda i: (i, 0)),
        ],
        out_specs=[
            pl.BlockSpec((1, 128), lambda i: (0, 0)),
            pl.BlockSpec((1, 128), lambda i: (0, 0)),
            pl.BlockSpec((1, 1), lambda i: (0, 0)),
        ],
        out_shape=[
            jax.ShapeDtypeStruct((1, 128), jnp.float32),
            jax.ShapeDtypeStruct((1, 128), jnp.float32),
            jax.ShapeDtypeStruct((1, 1), jnp.float32),
        ],
    )(G, X1p, C1, nbr, valid)


def _mlp_body(g_ref, x1s_ref, c1_ref, nbr_ref, val_ref, a_ref, b_ref,
              w2_ref, b2_ref, o_ref):
    pid = pl.program_id(0)
    gi = pid * BC + jax.lax.broadcasted_iota(jnp.int32, (BC, MAX_NB), 0)
    mask = (val_ref[...] != 0) & (nbr_ref[...] != gi)
    c1 = c1_ref[...]
    h1n = (g_ref[...] - c1[:, None, :]).reshape(BC * MAX_NB, 128)
    h1s = x1s_ref[...] - c1
    hall = jnp.concatenate([h1n, h1s], axis=0)
    hall = jnp.maximum(hall * a_ref[...] + b_ref[...], 0.0)
    h2 = jnp.dot(hall, w2_ref[...], preferred_element_type=jnp.float32) + b2_ref[...]
    h2n = h2[: BC * MAX_NB].reshape(BC, MAX_NB, 256)
    h2s = h2[BC * MAX_NB:]
    neg = jnp.float32(-jnp.inf)
    m3 = mask.astype(jnp.float32)[:, :, None]
    h2n = jnp.where(m3 != 0, h2n, neg)
    mx = jnp.max(h2n, axis=1)
    o_ref[...] = jnp.maximum(mx, h2s)


def _mlp(G, X1p, C1, nbr, valid, a, b, W2, b2):
    return pl.pallas_call(
        _mlp_body,
        grid=(M // BC,),
        in_specs=[
            pl.BlockSpec((BC, MAX_NB, 128), lambda i: (i, 0, 0)),
            pl.BlockSpec((BC, 128), lambda i: (i, 0)),
            pl.BlockSpec((BC, 128), lambda i: (i, 0)),
            pl.BlockSpec((BC, MAX_NB), lambda i: (i, 0)),
            pl.BlockSpec((BC, MAX_NB), lambda i: (i, 0)),
            pl.BlockSpec((1, 128), lambda i: (0, 0)),
            pl.BlockSpec((1, 128), lambda i: (0, 0)),
            pl.BlockSpec((128, 256), lambda i: (0, 0)),
            pl.BlockSpec((1, 256), lambda i: (0, 0)),
        ],
        out_specs=pl.BlockSpec((BC, 256), lambda i: (i, 0)),
        out_shape=jax.ShapeDtypeStruct((M, 256), jnp.float32),
    )(G, X1p, C1, nbr, valid, a, b, W2, b2[None, :])


def kernel(x, pos, batch, W1, b1, gamma, beta, W2, b2):
    idx = _fps(pos)
    pos_c = pos[idx]
    X1p = _x1p(x, pos, W1, b1)
    pos_c_pad = jnp.concatenate(
        [pos_c, jnp.broadcast_to(pos_c[:1], (M_PAD - M, 3))], axis=0)
    nbr, valid, G = _sc_select_gather(pos, pos_c_pad, X1p)
    C1 = (pos_c / RADIUS) @ W1[F:]
    sh, sh2, cnt = _stats(G, X1p, C1, nbr, valid)
    cnt = cnt[0, 0]
    mu = sh[0] / cnt
    var = sh2[0] / cnt - mu * mu
    a = gamma / jnp.sqrt(var + 1e-5)
    b = beta - mu * a
    out = _mlp(G, X1p, C1, nbr, valid, a[None, :], b[None, :], W2, b2)
    return (out, pos_c, batch[idx])
